# baseline (speedup 1.0000x reference)
import functools

import jax
import jax.numpy as jnp
from jax import lax
from jax.experimental import pallas as pl
from jax.experimental.pallas import tpu as pltpu
from jax.experimental.pallas import tpu_sc as plsc

_B = 4096
_H = 32
_W = 32
_LANES = 16
_IN_ROWS = _H * _W * _B // 128
_ROWS_PW = _IN_ROWS // 32
_CROWS = 128
_CHUNKS = _ROWS_PW // _CROWS


def _gate_body(ent_hbm, tf_hbm, tm_hbm, out_hbm,
               c_v0, m_v0, f_v0, c_v1, m_v1, f_v1,
               tf_v, tm_v, sin0, sin1, sout0, sout1):
    ncores = 2
    wid = lax.axis_index("s") * ncores + lax.axis_index("c")

    pltpu.sync_copy(tf_hbm, tf_v)
    pltpu.sync_copy(tm_hbm, tm_v)
    tf = tf_v[...]
    tm = tm_v[...]

    in_base = wid * _ROWS_PW
    out_base = wid * (3 * _ROWS_PW)

    out_bufs = ((c_v0, m_v0, f_v0), (c_v1, m_v1, f_v1))
    in_sems = (sin0, sin1)
    out_sems = (sout0, sout1)

    def start_in(g):
        p = g & 1
        return pltpu.async_copy(
            ent_hbm.at[pl.ds(in_base + g * _CROWS, _CROWS)], out_bufs[p][0],
            in_sems[p])

    def start_out(g):
        p = g & 1
        local = g * _CROWS
        return [
            pltpu.async_copy(
                out_bufs[p][j],
                out_hbm.at[pl.ds(out_base + j * _ROWS_PW + local, _CROWS)],
                out_sems[p])
            for j in range(3)
        ]

    h_in = {0: start_in(0)}
    h_out = {}
    for g in range(_CHUNKS):
        p = g & 1
        h_in.pop(g).wait()
        if g >= 2:
            for h in h_out[g - 2][1:]:
                h.wait()
        if g + 1 < _CHUNKS:
            if g >= 1:
                h_out[g - 1][0].wait()
            h_in[g + 1] = start_in(g + 1)

        cb, mb, fb = out_bufs[p]

        @plsc.parallel_loop(0, _CROWS * (128 // _LANES), unroll=8)
        def _(k):
            r = k // (128 // _LANES)
            c = pl.multiple_of((k % (128 // _LANES)) * _LANES, _LANES)
            v = cb[r, pl.ds(c, _LANES)]
            ci = (v <= tm).astype(jnp.int32)
            fi = (v > tf).astype(jnp.int32)
            cb[r, pl.ds(c, _LANES)] = plsc.bitcast(ci, jnp.float32)
            mb[r, pl.ds(c, _LANES)] = plsc.bitcast(1 - ci - fi, jnp.float32)
            fb[r, pl.ds(c, _LANES)] = plsc.bitcast(fi, jnp.float32)

        h_out[g] = start_out(g)

    for h in h_out[_CHUNKS - 2]:
        h.wait()
    for h in h_out[_CHUNKS - 1]:
        h.wait()


def kernel(entropy, threshold_fine, threshold_median):
    e = jnp.transpose(entropy, (1, 2, 0))
    e = e.reshape(_H, _W // 8, 8, _B // 128, 128)
    e = jnp.transpose(e, (0, 1, 3, 2, 4))
    ent_lin = e.reshape(_IN_ROWS, 128)

    tf = jnp.full((_LANES,), threshold_fine, jnp.float32)
    tm = jnp.full((_LANES,), threshold_median, jnp.float32)

    mesh = plsc.VectorSubcoreMesh(core_axis_name="c", subcore_axis_name="s")
    run = functools.partial(
        pl.kernel,
        out_type=jax.ShapeDtypeStruct((3 * _IN_ROWS, 128), jnp.float32),
        mesh=mesh,
        compiler_params=pltpu.CompilerParams(needs_layout_passes=False),
        scratch_types=[
            pltpu.VMEM((_CROWS, 128), jnp.float32),
            pltpu.VMEM((_CROWS, 128), jnp.float32),
            pltpu.VMEM((_CROWS, 128), jnp.float32),
            pltpu.VMEM((_CROWS, 128), jnp.float32),
            pltpu.VMEM((_CROWS, 128), jnp.float32),
            pltpu.VMEM((_CROWS, 128), jnp.float32),
            pltpu.VMEM((_LANES,), jnp.float32),
            pltpu.VMEM((_LANES,), jnp.float32),
            pltpu.SemaphoreType.DMA,
            pltpu.SemaphoreType.DMA,
            pltpu.SemaphoreType.DMA,
            pltpu.SemaphoreType.DMA,
        ],
    )(_gate_body)
    out = lax.bitcast_convert_type(run(ent_lin, tf, tm), jnp.int32)

    o = out.reshape(_H, 3, _W // 8, _B // 128, 8, 128)
    o = jnp.transpose(o, (3, 5, 0, 2, 4, 1))
    return o.reshape(_B, _H, _W, 3)


# speedup vs baseline: 1.6712x; 1.6712x over previous
import functools

import jax
import jax.numpy as jnp
from jax import lax
from jax.experimental import pallas as pl
from jax.experimental.pallas import tpu as pltpu
from jax.experimental.pallas import tpu_sc as plsc

_B = 4096
_H = 32
_W = 32
_LANES = 16
_IN_ROWS = _H * _W * _B // 128
_ROWS_PW = _IN_ROWS // 32
_CROWS = 64
_CHUNKS = _ROWS_PW // _CROWS


def _gate_body(ent_hbm, tf_hbm, tm_hbm, out_hbm,
               in0, in1, c0, m0, f0, c1, m1, f1,
               tf_v, tm_v, sin0, sin1, sout0, sout1):
    wid = lax.axis_index("s") * 2 + lax.axis_index("c")

    pltpu.sync_copy(tf_hbm, tf_v)
    pltpu.sync_copy(tm_hbm, tm_v)
    tf = tf_v[...]
    tm = tm_v[...]

    in_base = wid * _ROWS_PW
    out_base = wid * (3 * _ROWS_PW)

    bufs = ((in0, c0, m0, f0, sin0, sout0), (in1, c1, m1, f1, sin1, sout1))

    def start_in(g, p):
        ib, _, _, _, sin, _ = bufs[p]
        pltpu.async_copy(ent_hbm.at[pl.ds(in_base + g * _CROWS, _CROWS)],
                         ib, sin)

    def wait_in(p):
        ib, _, _, _, sin, _ = bufs[p]
        pltpu.make_async_copy(ent_hbm.at[pl.ds(0, _CROWS)], ib, sin).wait()

    def start_out(g, p):
        _, cb, mb, fb, _, sout = bufs[p]
        local = g * _CROWS
        for j, buf in enumerate((cb, mb, fb)):
            pltpu.async_copy(
                buf, out_hbm.at[pl.ds(out_base + j * _ROWS_PW + local, _CROWS)],
                sout)

    def wait_out(p):
        _, cb, mb, fb, _, sout = bufs[p]
        for buf in (cb, mb, fb):
            pltpu.make_async_copy(
                buf, out_hbm.at[pl.ds(out_base, _CROWS)], sout).wait()

    def compute(p):
        ib, cb, mb, fb, _, _ = bufs[p]

        @plsc.parallel_loop(0, _CROWS * (128 // _LANES), unroll=8)
        def _(k):
            r = k // (128 // _LANES)
            c = pl.multiple_of((k % (128 // _LANES)) * _LANES, _LANES)
            v = ib[r, pl.ds(c, _LANES)]
            ci = (v <= tm).astype(jnp.int32)
            fi = (v > tf).astype(jnp.int32)
            cb[r, pl.ds(c, _LANES)] = ci
            mb[r, pl.ds(c, _LANES)] = 1 - ci - fi
            fb[r, pl.ds(c, _LANES)] = fi

    def chunk(g, p, *, drain, prefetch):
        wait_in(p)
        if drain:
            wait_out(p)
        compute(p)
        start_out(g, p)
        if prefetch:
            start_in(g + 2, p)

    start_in(0, 0)
    start_in(1, 1)
    chunk(0, 0, drain=False, prefetch=True)
    chunk(1, 1, drain=False, prefetch=True)

    def pair(i, carry):
        chunk(2 * i, 0, drain=True, prefetch=True)
        chunk(2 * i + 1, 1, drain=True, prefetch=True)
        return carry

    lax.fori_loop(1, _CHUNKS // 2 - 1, pair, jnp.int32(0))

    chunk(_CHUNKS - 2, 0, drain=True, prefetch=False)
    chunk(_CHUNKS - 1, 1, drain=True, prefetch=False)
    wait_out(0)
    wait_out(1)


def kernel(entropy, threshold_fine, threshold_median):
    e = jnp.transpose(entropy, (1, 2, 0))
    e = e.reshape(_H, _W // 8, 8, _B // 128, 128)
    e = jnp.transpose(e, (0, 1, 3, 2, 4))
    ent_lin = e.reshape(_IN_ROWS, 128)

    tf = jnp.full((_LANES,), threshold_fine, jnp.float32)
    tm = jnp.full((_LANES,), threshold_median, jnp.float32)

    mesh = plsc.VectorSubcoreMesh(core_axis_name="c", subcore_axis_name="s")
    run = functools.partial(
        pl.kernel,
        out_type=jax.ShapeDtypeStruct((3 * _IN_ROWS, 128), jnp.int32),
        mesh=mesh,
        compiler_params=pltpu.CompilerParams(needs_layout_passes=False),
        scratch_types=[
            pltpu.VMEM((_CROWS, 128), jnp.float32),
            pltpu.VMEM((_CROWS, 128), jnp.float32),
            pltpu.VMEM((_CROWS, 128), jnp.int32),
            pltpu.VMEM((_CROWS, 128), jnp.int32),
            pltpu.VMEM((_CROWS, 128), jnp.int32),
            pltpu.VMEM((_CROWS, 128), jnp.int32),
            pltpu.VMEM((_CROWS, 128), jnp.int32),
            pltpu.VMEM((_CROWS, 128), jnp.int32),
            pltpu.VMEM((_LANES,), jnp.float32),
            pltpu.VMEM((_LANES,), jnp.float32),
            pltpu.SemaphoreType.DMA,
            pltpu.SemaphoreType.DMA,
            pltpu.SemaphoreType.DMA,
            pltpu.SemaphoreType.DMA,
        ],
    )(_gate_body)
    out = run(ent_lin, tf, tm)

    o = out.reshape(_H, 3, _W // 8, _B // 128, 8, 128)
    o = jnp.transpose(o, (3, 5, 0, 2, 4, 1))
    return o.reshape(_B, _H, _W, 3)


# speedup vs baseline: 1.6936x; 1.0134x over previous
import functools

import jax
import jax.numpy as jnp
from jax import lax
from jax.experimental import pallas as pl
from jax.experimental.pallas import tpu as pltpu
from jax.experimental.pallas import tpu_sc as plsc

_B = 4096
_H = 32
_W = 32
_LANES = 16
_IN_ROWS = _H * _W * _B // 128
_ROWS_PW = _IN_ROWS // 32
_CROWS = 64
_CHUNKS = _ROWS_PW // _CROWS


def _gate_body(ent_hbm, tf_hbm, tm_hbm, out_hbm,
               in0, in1, c0, m0, f0, c1, m1, f1,
               tf_v, tm_v, sin0, sin1, sout0, sout1):
    wid = lax.axis_index("s") * 2 + lax.axis_index("c")

    pltpu.sync_copy(tf_hbm, tf_v)
    pltpu.sync_copy(tm_hbm, tm_v)
    tf = tf_v[...]
    tm = tm_v[...]

    in_base = wid * _ROWS_PW
    out_base = wid * (3 * _ROWS_PW)

    bufs = ((in0, c0, m0, f0, sin0, sout0), (in1, c1, m1, f1, sin1, sout1))

    def start_in(g, p):
        ib, _, _, _, sin, _ = bufs[p]
        pltpu.async_copy(ent_hbm.at[pl.ds(in_base + g * _CROWS, _CROWS)],
                         ib, sin)

    def wait_in(p):
        ib, _, _, _, sin, _ = bufs[p]
        pltpu.make_async_copy(ent_hbm.at[pl.ds(0, _CROWS)], ib, sin).wait()

    def start_out(g, p):
        _, cb, mb, fb, _, sout = bufs[p]
        local = g * _CROWS
        for j, buf in enumerate((cb, mb, fb)):
            pltpu.async_copy(
                buf, out_hbm.at[pl.ds(out_base + j * _ROWS_PW + local, _CROWS)],
                sout)

    def wait_out(p):
        _, cb, mb, fb, _, sout = bufs[p]
        for buf in (cb, mb, fb):
            pltpu.make_async_copy(
                buf, out_hbm.at[pl.ds(out_base, _CROWS)], sout).wait()

    def compute(p):
        ib, cb, mb, fb, _, _ = bufs[p]

        @plsc.parallel_loop(0, _CROWS * (128 // _LANES), unroll=8)
        def _(k):
            r = k // (128 // _LANES)
            c = pl.multiple_of((k % (128 // _LANES)) * _LANES, _LANES)
            v = ib[r, pl.ds(c, _LANES)]
            ci = (v <= tm).astype(jnp.int32)
            fi = (v > tf).astype(jnp.int32)
            cb[r, pl.ds(c, _LANES)] = ci
            mb[r, pl.ds(c, _LANES)] = 1 - ci - fi
            fb[r, pl.ds(c, _LANES)] = fi

    start_in(0, 0)
    start_in(1, 1)

    def pair(i, carry):
        for p in range(2):
            g = 2 * i + p
            wait_in(p)

            @pl.when(i >= 1)
            def _():
                wait_out(p)

            compute(p)
            start_out(g, p)

            @pl.when(i < _CHUNKS // 2 - 1)
            def _():
                start_in(g + 2, p)

        return carry

    lax.fori_loop(0, _CHUNKS // 2, pair, jnp.int32(0))
    wait_out(0)
    wait_out(1)


def kernel(entropy, threshold_fine, threshold_median):
    e = jnp.transpose(entropy, (1, 2, 0))
    e = e.reshape(_H, _W // 8, 8, _B // 128, 128)
    e = jnp.transpose(e, (0, 1, 3, 2, 4))
    ent_lin = e.reshape(_IN_ROWS, 128)

    tf = jnp.full((_LANES,), threshold_fine, jnp.float32)
    tm = jnp.full((_LANES,), threshold_median, jnp.float32)

    mesh = plsc.VectorSubcoreMesh(core_axis_name="c", subcore_axis_name="s")
    run = functools.partial(
        pl.kernel,
        out_type=jax.ShapeDtypeStruct((3 * _IN_ROWS, 128), jnp.int32),
        mesh=mesh,
        compiler_params=pltpu.CompilerParams(needs_layout_passes=False),
        scratch_types=[
            pltpu.VMEM((_CROWS, 128), jnp.float32),
            pltpu.VMEM((_CROWS, 128), jnp.float32),
            pltpu.VMEM((_CROWS, 128), jnp.int32),
            pltpu.VMEM((_CROWS, 128), jnp.int32),
            pltpu.VMEM((_CROWS, 128), jnp.int32),
            pltpu.VMEM((_CROWS, 128), jnp.int32),
            pltpu.VMEM((_CROWS, 128), jnp.int32),
            pltpu.VMEM((_CROWS, 128), jnp.int32),
            pltpu.VMEM((_LANES,), jnp.float32),
            pltpu.VMEM((_LANES,), jnp.float32),
            pltpu.SemaphoreType.DMA,
            pltpu.SemaphoreType.DMA,
            pltpu.SemaphoreType.DMA,
            pltpu.SemaphoreType.DMA,
        ],
    )(_gate_body)
    out = run(ent_lin, tf, tm)

    o = out.reshape(_H, 3, _W // 8, _B // 128, 8, 128)
    o = jnp.transpose(o, (3, 5, 0, 2, 4, 1))
    return o.reshape(_B, _H, _W, 3)
